# baseline (speedup 1.0000x reference)
import functools

import jax
import jax.numpy as jnp
from jax import lax
from jax.experimental import pallas as pl
from jax.experimental.pallas import tpu as pltpu
from jax.experimental.pallas import tpu_sc as plsc

NC = 2
NS = 16
NW = NC * NS
C = 800


@functools.partial(jax.jit, static_argnames=("n_chunks",))
def _embed_lookup(idx, table, pos, *, n_chunks):
    n, = idx.shape
    _, d = table.shape
    s, _ = pos.shape
    per_w = n // NW
    mesh = plsc.VectorSubcoreMesh(core_axis_name="c", subcore_axis_name="s")

    @functools.partial(
        pl.kernel,
        out_type=jax.ShapeDtypeStruct((n, d), jnp.float32),
        mesh=mesh,
        scratch_types=[
            pltpu.VMEM((C,), jnp.int32),
            pltpu.VMEM((C,), jnp.int32),
            pltpu.VMEM((C, d), jnp.float32),
            pltpu.VMEM((2 * s, d), jnp.float32),
            pltpu.VMEM((1, d), jnp.float32),
            pltpu.HBM((NC * 2 * s, d), jnp.float32),
            pltpu.SemaphoreType.DMA,
        ],
        compiler_params=pltpu.CompilerParams(use_tc_tiling_on_sc=False),
    )
    def body(idx_hbm, table_hbm, pos_hbm, out_hbm,
             idx_v, cidx_v, rows_v, combo_v, r0_v, combo_hbm, sem):
        cid = lax.axis_index("c")
        sid = lax.axis_index("s")
        wid = sid * NC + cid
        base = wid * per_w
        combo_base = cid * 2 * s

        @pl.when(sid == 0)
        def _build():
            pltpu.sync_copy(table_hbm.at[pl.ds(0, 1)], r0_v)
            pltpu.sync_copy(pos_hbm, combo_v.at[pl.ds(0, s)])
            pltpu.sync_copy(pos_hbm, combo_v.at[pl.ds(s, s)])
            for h in range(d // 16):
                r0h = r0_v[0, pl.ds(h * 16, 16)]
                for j in range(s):
                    combo_v[s + j, pl.ds(h * 16, 16)] = (
                        combo_v[s + j, pl.ds(h * 16, 16)] - r0h)
            pltpu.sync_copy(combo_v, combo_hbm.at[pl.ds(combo_base, 2 * s)])

        plsc.subcore_barrier()

        iota = lax.iota(jnp.int32, 16)

        def build_cidx(k, carry):
            o = pl.multiple_of(k * 16, 16)
            v = idx_v[pl.ds(o, 16)]
            sp = lax.rem(iota + o, s)
            pad = jnp.where(v == 0, s, 0).astype(jnp.int32)
            cidx_v[pl.ds(o, 16)] = sp + pad + combo_base
            return carry

        for g in range(n_chunks):
            off = base + g * C
            pltpu.sync_copy(idx_hbm.at[pl.ds(off, C)], idx_v)
            lax.fori_loop(0, C // 16, build_cidx, 0)
            pltpu.async_copy(table_hbm.at[idx_v], rows_v, sem).wait()
            pltpu.async_copy(combo_hbm.at[cidx_v], rows_v, sem, add=True).wait()
            pltpu.sync_copy(rows_v, out_hbm.at[pl.ds(off, C)])

    return body(idx, table, pos)


def kernel(x, embedding_table, pos_embeddings):
    b, s = x.shape
    _, d = embedding_table.shape
    n = b * s
    idx = x.reshape(n).astype(jnp.int32)
    n_chunks = n // (NW * C)
    out = _embed_lookup(idx, embedding_table, pos_embeddings,
                        n_chunks=n_chunks)
    return out.reshape(b, s, d)


# speedup vs baseline: 1.0028x; 1.0028x over previous
import functools

import jax
import jax.numpy as jnp
from jax import lax
from jax.experimental import pallas as pl
from jax.experimental.pallas import tpu as pltpu
from jax.experimental.pallas import tpu_sc as plsc

NC = 2
NS = 16
NW = NC * NS
C = 800


@functools.partial(jax.jit, static_argnames=("n_chunks",))
def _embed_lookup(idx, table, pos, *, n_chunks):
    n, = idx.shape
    _, d = table.shape
    s, _ = pos.shape
    per_w = n // NW
    mesh = plsc.VectorSubcoreMesh(core_axis_name="c", subcore_axis_name="s")

    @functools.partial(
        pl.kernel,
        out_type=jax.ShapeDtypeStruct((n, d), jnp.float32),
        mesh=mesh,
        scratch_types=[
            pltpu.VMEM((C,), jnp.int32),
            pltpu.VMEM((C,), jnp.int32),
            pltpu.VMEM((C,), jnp.int32),
            pltpu.VMEM((C,), jnp.int32),
            pltpu.VMEM((C, d), jnp.float32),
            pltpu.VMEM((C, d), jnp.float32),
            pltpu.VMEM((2 * s, d), jnp.float32),
            pltpu.VMEM((1, d), jnp.float32),
            pltpu.HBM((NC * 2 * s, d), jnp.float32),
            pltpu.SemaphoreType.DMA,
            pltpu.SemaphoreType.DMA,
            pltpu.SemaphoreType.DMA,
            pltpu.SemaphoreType.DMA,
            pltpu.SemaphoreType.DMA,
            pltpu.SemaphoreType.DMA,
        ],
        compiler_params=pltpu.CompilerParams(use_tc_tiling_on_sc=False),
    )
    def body(idx_hbm, table_hbm, pos_hbm, out_hbm,
             idx_v0, idx_v1, cidx_v0, cidx_v1, rows_v0, rows_v1,
             combo_v, r0_v, combo_hbm,
             semi0, semi1, semg0, semg1, semo0, semo1):
        cid = lax.axis_index("c")
        sid = lax.axis_index("s")
        wid = sid * NC + cid
        base = wid * per_w
        combo_base = cid * 2 * s

        @pl.when(sid == 0)
        def _build():
            pltpu.sync_copy(table_hbm.at[pl.ds(0, 1)], r0_v)
            pltpu.sync_copy(pos_hbm, combo_v.at[pl.ds(0, s)])
            pltpu.sync_copy(pos_hbm, combo_v.at[pl.ds(s, s)])
            for h in range(d // 16):
                r0h = r0_v[0, pl.ds(h * 16, 16)]
                for j in range(s):
                    combo_v[s + j, pl.ds(h * 16, 16)] = (
                        combo_v[s + j, pl.ds(h * 16, 16)] - r0h)
            pltpu.sync_copy(combo_v, combo_hbm.at[pl.ds(combo_base, 2 * s)])

        plsc.subcore_barrier()

        iota = lax.iota(jnp.int32, 16)
        idx_v = [idx_v0, idx_v1]
        cidx_v = [cidx_v0, cidx_v1]
        rows_v = [rows_v0, rows_v1]
        semi = [semi0, semi1]
        semg = [semg0, semg1]
        semo = [semo0, semo1]

        def build_cidx(buf):
            def f(k, carry):
                o = pl.multiple_of(k * 16, 16)
                v = idx_v[buf][pl.ds(o, 16)]
                sp = lax.rem(iota + o, s)
                pad = jnp.where(v == 0, s, 0).astype(jnp.int32)
                cidx_v[buf][pl.ds(o, 16)] = sp + pad + combo_base
                return carry
            lax.fori_loop(0, C // 16, f, 0)

        def off(g):
            return base + g * C

        G = n_chunks
        tg = [None, None]
        ca = [None, None]
        ou = [None, None]
        ic = [None, None]

        pltpu.sync_copy(idx_hbm.at[pl.ds(off(0), C)], idx_v[0])
        build_cidx(0)
        tg[0] = pltpu.async_copy(table_hbm.at[idx_v[0]], rows_v[0], semg[0])
        if G > 1:
            ic[1] = pltpu.async_copy(idx_hbm.at[pl.ds(off(1), C)],
                                     idx_v[1], semi[1])

        for g in range(G):
            b = g % 2
            nb = 1 - b
            tg[b].wait()
            ca[b] = pltpu.async_copy(combo_hbm.at[cidx_v[b]], rows_v[b],
                                     semg[b], add=True)
            if g + 1 < G:
                ic[nb].wait()
                build_cidx(nb)
                if g + 2 < G:
                    ic[b] = pltpu.async_copy(idx_hbm.at[pl.ds(off(g + 2), C)],
                                             idx_v[b], semi[b])
            ca[b].wait()
            ou[b] = pltpu.async_copy(rows_v[b], out_hbm.at[pl.ds(off(g), C)],
                                     semo[b])
            if g + 1 < G:
                if g >= 1:
                    ou[nb].wait()
                tg[nb] = pltpu.async_copy(table_hbm.at[idx_v[nb]],
                                          rows_v[nb], semg[nb])

        if G > 1:
            ou[(G - 2) % 2].wait()
        ou[(G - 1) % 2].wait()

    return body(idx, table, pos)


def kernel(x, embedding_table, pos_embeddings):
    b, s = x.shape
    _, d = embedding_table.shape
    n = b * s
    idx = x.reshape(n).astype(jnp.int32)
    n_chunks = n // (NW * C)
    out = _embed_lookup(idx, embedding_table, pos_embeddings,
                        n_chunks=n_chunks)
    return out.reshape(b, s, d)


# speedup vs baseline: 1.0879x; 1.0849x over previous
import functools

import jax
import jax.numpy as jnp
from jax import lax
from jax.experimental import pallas as pl
from jax.experimental.pallas import tpu as pltpu
from jax.experimental.pallas import tpu_sc as plsc

NC = 2
NS = 16
NW = NC * NS
C = 800


@functools.partial(jax.jit, static_argnames=("n_chunks",))
def _embed_lookup(idx, table, pos, *, n_chunks):
    n, = idx.shape
    _, d = table.shape
    s, _ = pos.shape
    per_w = n // NW
    mesh = plsc.VectorSubcoreMesh(core_axis_name="c", subcore_axis_name="s")

    @functools.partial(
        pl.kernel,
        out_type=jax.ShapeDtypeStruct((n, d), jnp.float32),
        mesh=mesh,
        scratch_types=[
            pltpu.VMEM((C,), jnp.int32),
            pltpu.VMEM((C,), jnp.int32),
            pltpu.VMEM((C,), jnp.int32),
            pltpu.VMEM((C,), jnp.int32),
            pltpu.VMEM((C, d), jnp.float32),
            pltpu.VMEM((C, d), jnp.float32),
            pltpu.VMEM((2 * s, d), jnp.float32),
            pltpu.VMEM((1, d), jnp.float32),
            pltpu.HBM((NC * 16 * 2 * s, d), jnp.float32),
            pltpu.SemaphoreType.DMA,
            pltpu.SemaphoreType.DMA,
            pltpu.SemaphoreType.DMA,
            pltpu.SemaphoreType.DMA,
            pltpu.SemaphoreType.DMA,
            pltpu.SemaphoreType.DMA,
        ],
        compiler_params=pltpu.CompilerParams(use_tc_tiling_on_sc=False),
    )
    def body(idx_hbm, table_hbm, pos_hbm, out_hbm,
             idx_v0, idx_v1, cidx_v0, cidx_v1, rows_v0, rows_v1,
             combo_v, r0_v, combo_hbm,
             semi0, semi1, semg0, semg1, semo0, semo1):
        cid = lax.axis_index("c")
        sid = lax.axis_index("s")
        wid = sid * NC + cid
        base = wid * per_w
        combo_base = cid * 16 * 2 * s

        @pl.when(sid == 0)
        def _build():
            pltpu.sync_copy(table_hbm.at[pl.ds(0, 1)], r0_v)
            pltpu.sync_copy(pos_hbm, combo_v.at[pl.ds(0, s)])
            pltpu.sync_copy(pos_hbm, combo_v.at[pl.ds(s, s)])
            for h in range(d // 16):
                r0h = r0_v[0, pl.ds(h * 16, 16)]
                for j in range(s):
                    combo_v[s + j, pl.ds(h * 16, 16)] = (
                        combo_v[s + j, pl.ds(h * 16, 16)] - r0h)
            for r in range(16):
                pltpu.sync_copy(
                    combo_v,
                    combo_hbm.at[pl.ds(combo_base + r * 2 * s, 2 * s)])

        plsc.subcore_barrier()

        iota = lax.iota(jnp.int32, 16)
        idx_v = [idx_v0, idx_v1]
        cidx_v = [cidx_v0, cidx_v1]
        rows_v = [rows_v0, rows_v1]
        semi = [semi0, semi1]
        semg = [semg0, semg1]
        semo = [semo0, semo1]

        def build_cidx(buf):
            def f(k, carry):
                o = pl.multiple_of(k * 16, 16)
                v = idx_v[buf][pl.ds(o, 16)]
                sp = lax.rem(iota + o, s)
                pad = jnp.where(v == 0, s, 0).astype(jnp.int32)
                rep = lax.rem(k + wid, 16) * (2 * s)
                cidx_v[buf][pl.ds(o, 16)] = sp + pad + combo_base + rep
                return carry
            lax.fori_loop(0, C // 16, f, 0)

        def off(g):
            return base + g * C

        G = n_chunks
        tg = [None, None]
        ca = [None, None]
        ou = [None, None]
        ic = [None, None]

        pltpu.sync_copy(idx_hbm.at[pl.ds(off(0), C)], idx_v[0])
        build_cidx(0)
        tg[0] = pltpu.async_copy(table_hbm.at[idx_v[0]], rows_v[0], semg[0])
        if G > 1:
            ic[1] = pltpu.async_copy(idx_hbm.at[pl.ds(off(1), C)],
                                     idx_v[1], semi[1])

        for g in range(G):
            b = g % 2
            nb = 1 - b
            tg[b].wait()
            ca[b] = pltpu.async_copy(combo_hbm.at[cidx_v[b]], rows_v[b],
                                     semg[b], add=True)
            if g + 1 < G:
                ic[nb].wait()
                build_cidx(nb)
                if g + 2 < G:
                    ic[b] = pltpu.async_copy(idx_hbm.at[pl.ds(off(g + 2), C)],
                                             idx_v[b], semi[b])
            ca[b].wait()
            ou[b] = pltpu.async_copy(rows_v[b], out_hbm.at[pl.ds(off(g), C)],
                                     semo[b])
            if g + 1 < G:
                if g >= 1:
                    ou[nb].wait()
                tg[nb] = pltpu.async_copy(table_hbm.at[idx_v[nb]],
                                          rows_v[nb], semg[nb])

        if G > 1:
            ou[(G - 2) % 2].wait()
        ou[(G - 1) % 2].wait()

    return body(idx, table, pos)


def kernel(x, embedding_table, pos_embeddings):
    b, s = x.shape
    _, d = embedding_table.shape
    n = b * s
    idx = x.reshape(n).astype(jnp.int32)
    n_chunks = n // (NW * C)
    out = _embed_lookup(idx, embedding_table, pos_embeddings,
                        n_chunks=n_chunks)
    return out.reshape(b, s, d)


# speedup vs baseline: 1.2220x; 1.1232x over previous
import functools

import jax
import jax.numpy as jnp
from jax import lax
from jax.experimental import pallas as pl
from jax.experimental.pallas import tpu as pltpu
from jax.experimental.pallas import tpu_sc as plsc

NC = 2
NS = 16
NW = NC * NS
C = 800


@functools.partial(jax.jit, static_argnames=("n_chunks",))
def _embed_lookup(idx, table, pos, *, n_chunks):
    n, = idx.shape
    _, d = table.shape
    s, _ = pos.shape
    per_w = n // NW
    mesh = plsc.VectorSubcoreMesh(core_axis_name="c", subcore_axis_name="s")

    @functools.partial(
        pl.kernel,
        out_type=jax.ShapeDtypeStruct((s, d, n // s), jnp.float32),
        mesh=mesh,
        scratch_types=[
            pltpu.VMEM((C,), jnp.int32),
            pltpu.VMEM((C,), jnp.int32),
            pltpu.VMEM((C,), jnp.int32),
            pltpu.VMEM((C,), jnp.int32),
            pltpu.VMEM((C, d), jnp.float32),
            pltpu.VMEM((C, d), jnp.float32),
            pltpu.VMEM((50, 32, 16), jnp.float32),
            pltpu.VMEM((50, 32, 16), jnp.float32),
            pltpu.VMEM((2 * s, d), jnp.float32),
            pltpu.VMEM((1, d), jnp.float32),
            pltpu.HBM((NC * 16 * 2 * s, d), jnp.float32),
            pltpu.SemaphoreType.DMA,
            pltpu.SemaphoreType.DMA,
            pltpu.SemaphoreType.DMA,
            pltpu.SemaphoreType.DMA,
            pltpu.SemaphoreType.DMA,
            pltpu.SemaphoreType.DMA,
        ],
        compiler_params=pltpu.CompilerParams(use_tc_tiling_on_sc=False, needs_layout_passes=False),
    )
    def body(idx_hbm, table_hbm, pos_hbm, out_hbm,
             idx_v0, idx_v1, cidx_v0, cidx_v1, rows_v0, rows_v1,
             stage_v0, stage_v1, combo_v, r0_v, combo_hbm,
             semi0, semi1, semg0, semg1, semo0, semo1):
        cid = lax.axis_index("c")
        sid = lax.axis_index("s")
        wid = sid * NC + cid
        base = wid * per_w
        combo_base = cid * 16 * 2 * s

        @pl.when(sid == 0)
        def _build():
            pltpu.sync_copy(table_hbm.at[pl.ds(0, 1)], r0_v)
            pltpu.sync_copy(pos_hbm, combo_v.at[pl.ds(0, s)])
            pltpu.sync_copy(pos_hbm, combo_v.at[pl.ds(s, s)])
            for h in range(d // 16):
                r0h = r0_v[0, pl.ds(h * 16, 16)]
                for j in range(s):
                    combo_v[s + j, pl.ds(h * 16, 16)] = (
                        combo_v[s + j, pl.ds(h * 16, 16)] - r0h)
            for r in range(16):
                pltpu.sync_copy(
                    combo_v,
                    combo_hbm.at[pl.ds(combo_base + r * 2 * s, 2 * s)])

        plsc.subcore_barrier()

        iota = lax.iota(jnp.int32, 16)
        idx_v = [idx_v0, idx_v1]
        cidx_v = [cidx_v0, cidx_v1]
        rows_v = [rows_v0, rows_v1]
        stage_v = [stage_v0, stage_v1]

        nb16 = C // s

        def transpose_chunk(buf):
            def f(sj, carry):
                pvec = iota * s + sj
                for c in range(d):
                    csp = jnp.full((16,), c, jnp.int32)
                    vals = plsc.load_gather(rows_v[buf], [pvec, csp])
                    stage_v[buf][sj, c, pl.ds(0, 16)] = vals
                return carry
            lax.fori_loop(0, s, f, 0)
        semi = [semi0, semi1]
        semg = [semg0, semg1]
        semo = [semo0, semo1]

        def build_cidx(buf):
            def f(k, carry):
                o = pl.multiple_of(k * 16, 16)
                v = idx_v[buf][pl.ds(o, 16)]
                sp = lax.rem(iota + o, s)
                pad = jnp.where(v == 0, s, 0).astype(jnp.int32)
                rep = lax.rem(k + wid, 16) * (2 * s)
                cidx_v[buf][pl.ds(o, 16)] = sp + pad + combo_base + rep
                return carry
            lax.fori_loop(0, C // 16, f, 0)

        def off(g):
            return base + g * C

        G = n_chunks
        tg = [None, None]
        ca = [None, None]
        ou = [None, None]
        ic = [None, None]

        pltpu.sync_copy(idx_hbm.at[pl.ds(off(0), C)], idx_v[0])
        build_cidx(0)
        tg[0] = pltpu.async_copy(table_hbm.at[idx_v[0]], rows_v[0], semg[0])
        if G > 1:
            ic[1] = pltpu.async_copy(idx_hbm.at[pl.ds(off(1), C)],
                                     idx_v[1], semi[1])

        for g in range(G):
            b = g % 2
            nb = 1 - b
            tg[b].wait()
            ca[b] = pltpu.async_copy(combo_hbm.at[cidx_v[b]], rows_v[b],
                                     semg[b], add=True)
            if g + 1 < G:
                ic[nb].wait()
                build_cidx(nb)
                if g + 2 < G:
                    ic[b] = pltpu.async_copy(idx_hbm.at[pl.ds(off(g + 2), C)],
                                             idx_v[b], semi[b])
            ca[b].wait()
            if g >= 2:
                ou[b].wait()
            transpose_chunk(b)
            bo = wid * (per_w // s) + g * nb16
            ou[b] = pltpu.async_copy(stage_v[b],
                                     out_hbm.at[:, :, pl.ds(bo, nb16)],
                                     semo[b])
            if g + 1 < G:
                tg[nb] = pltpu.async_copy(table_hbm.at[idx_v[nb]],
                                          rows_v[nb], semg[nb])

        if G > 1:
            ou[(G - 2) % 2].wait()
        ou[(G - 1) % 2].wait()
        _ = sid

    return body(idx, table, pos)


def kernel(x, embedding_table, pos_embeddings):
    b, s = x.shape
    _, d = embedding_table.shape
    n = b * s
    idx = x.reshape(n).astype(jnp.int32)
    n_chunks = n // (NW * C)
    out3 = _embed_lookup(idx, embedding_table, pos_embeddings,
                         n_chunks=n_chunks)
    return out3.transpose(2, 0, 1)
